# baseline (speedup 1.0000x reference)
import functools

import jax
import jax.numpy as jnp
from jax import lax
from jax.experimental import pallas as pl
from jax.experimental.pallas import tpu as pltpu
from jax.experimental.pallas import tpu_sc as plsc

_NS = 10000
_NR = 10000
_E = 160000
_D = 256
_EF = 16

_NC = 2
_NSUB = 16
_NW = _NC * _NSUB

_NBG = 5
_NBS = 2
_GC = 40
_GN = _E // _NW // _GC
_SC = 80
_SN = _E // _NSUB // _SC
_HALF = _D // _NC
_DP = _D // 2


@functools.cache
def _mesh():
    return plsc.VectorSubcoreMesh(core_axis_name="c", subcore_axis_name="s")


def _pipe(n, nb, fire_a, drain_a, fire_b, drain_b):
    if n % nb:
        assert nb == 2 and n % 2 == 1
        fire_a(0, 0)
        fire_a(1, 1)

        @pl.loop(2, n - 1, step=2)
        def _pipe_odd(ch0):
            for b in (0, 1):
                ch = ch0 + b
                drain_a(b)
                fire_b(ch - 2, b)
                drain_b(b)
                fire_a(ch, b)

        drain_a(0)
        fire_b(n - 3, 0)
        drain_b(0)
        fire_a(n - 1, 0)
        drain_a(1)
        fire_b(n - 2, 1)
        drain_a(0)
        fire_b(n - 1, 0)
        drain_b(1)
        drain_b(0)
        return

    assert n >= 2 * nb
    for b in range(nb):
        fire_a(b, b)

    @pl.loop(nb, n, step=nb)
    def _body(ch0):
        for b in range(nb):
            drain_a(b)
            fire_b(ch0 - nb + b, b)
        for b in range(nb):
            drain_b(b)
            fire_a(ch0 + b, b)

    for b in range(nb):
        drain_a(b)
        fire_b(n - nb + b, b)
    for b in range(nb):
        drain_b(b)


def _gather_body(table_hbm, idx_hbm, out_hbm, idx_v, buf, *sems):
    c = lax.axis_index("c")
    s = lax.axis_index("s")
    wid = s * _NC + c
    e0 = wid * (_GN * _GC)
    pltpu.sync_copy(idx_hbm.at[wid], idx_v)
    gsem = sems[:_NBG]
    wsem = sems[_NBG:]

    def fire_gather(ch, b):
        pltpu.async_copy(table_hbm.at[idx_v.at[ch]], buf.at[b], gsem[b])

    def drain_gather(b):
        pltpu.make_async_copy(table_hbm.at[idx_v.at[0]], buf.at[b],
                              gsem[b]).wait()

    def fire_wb(ch, b):
        base = pl.multiple_of(e0 + ch * _GC, 8)
        pltpu.async_copy(buf.at[b], out_hbm.at[pl.ds(base, _GC)], wsem[b])

    def drain_wb(b):
        pltpu.make_async_copy(buf.at[b], out_hbm.at[pl.ds(0, _GC)],
                              wsem[b]).wait()

    _pipe(_GN, _NBG, fire_gather, drain_gather, fire_wb, drain_wb)


def _sc_gather(table, idx3):
    return pl.kernel(
        _gather_body,
        out_type=jax.ShapeDtypeStruct((_E, _DP), jnp.float32),
        mesh=_mesh(),
        scratch_types=[
            pltpu.VMEM((_GN, _GC), jnp.int32),
            pltpu.VMEM((_NBG, _GC, _DP), jnp.float32),
        ] + [pltpu.SemaphoreType.DMA] * (2 * _NBG),
        name="sc_gather_rows",
    )(table, idx3)


def _scatter_body(msgs_hbm, idx_hbm, base_hbm, out_hbm, didx_v, mbuf, acc_sh,
                  *sems):
    c = lax.axis_index("c")
    s = lax.axis_index("s")
    col0 = pl.multiple_of(c * _HALF, _HALF)
    lsem = sems[:_NBS]
    ssem = sems[_NBS:]

    pltpu.sync_copy(idx_hbm.at[s], didx_v)
    n_rows = base_hbm.shape[0]
    nrc = n_rows // _SC
    my_rows = (nrc - s + _NSUB - 1) // _NSUB

    def init_body(i, carry):
        r0 = pl.multiple_of((s + i * _NSUB) * _SC, 8)
        pltpu.sync_copy(base_hbm.at[pl.ds(r0, _SC), pl.ds(col0, _HALF)],
                        mbuf.at[0])
        pltpu.sync_copy(mbuf.at[0], acc_sh.at[pl.ds(r0, _SC)])
        return carry

    lax.fori_loop(0, my_rows, init_body, 0)
    plsc.subcore_barrier()

    e0 = s * (_E // _NSUB)

    def fire_load(ch, b):
        base = pl.multiple_of(e0 + ch * _SC, 8)
        pltpu.async_copy(msgs_hbm.at[pl.ds(base, _SC), pl.ds(col0, _HALF)],
                         mbuf.at[b], lsem[b])

    def drain_load(b):
        pltpu.make_async_copy(
            msgs_hbm.at[pl.ds(0, _SC), pl.ds(col0, _HALF)], mbuf.at[b],
            lsem[b]).wait()

    def fire_scat(ch, b):
        pltpu.async_copy(mbuf.at[b], acc_sh.at[didx_v.at[ch]], ssem[b],
                         add=True)

    def drain_scat(b):
        pltpu.make_async_copy(mbuf.at[b], acc_sh.at[didx_v.at[0]],
                              ssem[b]).wait()

    _pipe(_SN, _NBS, fire_load, drain_load, fire_scat, drain_scat)
    plsc.subcore_barrier()

    def out_body(i, carry):
        r0 = pl.multiple_of((s + i * _NSUB) * _SC, 8)
        pltpu.sync_copy(acc_sh.at[pl.ds(r0, _SC)], mbuf.at[0])
        pltpu.sync_copy(mbuf.at[0],
                        out_hbm.at[pl.ds(r0, _SC), pl.ds(col0, _HALF)])
        return carry

    lax.fori_loop(0, my_rows, out_body, 0)


def _sc_scatter_add(msgs, idx3, base):
    n = base.shape[0]
    return pl.kernel(
        _scatter_body,
        out_type=jax.ShapeDtypeStruct((n, _D), jnp.float32),
        mesh=_mesh(),
        scratch_types=[
            pltpu.VMEM((_SN, _SC), jnp.int32),
            pltpu.VMEM((_NBS, _SC, _HALF), jnp.float32),
            pltpu.VMEM_SHARED((n, _HALF), jnp.float32),
        ] + [pltpu.SemaphoreType.DMA] * (2 * _NBS),
        name="sc_scatter_add",
    )(msgs, idx3, base)


def _layernorm(x, g, b, eps=1e-5):
    mu = jnp.mean(x, axis=-1, keepdims=True)
    var = jnp.mean((x - mu) ** 2, axis=-1, keepdims=True)
    return (x - mu) / jnp.sqrt(var + eps) * g + b


def _mlp_block(g_ref, f_ref, w1h_ref, w1f_ref, b1_ref, w2_ref, b2_ref,
               gam_ref, bet_ref, o_ref, *, fuse_ln):
    bf = jnp.bfloat16
    xp = g_ref[...]
    x = pltpu.bitcast(xp, bf).reshape(xp.shape[0], _D)
    if fuse_ln:
        x = _layernorm(x.astype(jnp.float32), gam_ref[...],
                       bet_ref[...]).astype(bf)
    a = (jnp.dot(x, w1h_ref[...].astype(bf),
                 preferred_element_type=jnp.float32)
         + jnp.dot(f_ref[...].astype(bf), w1f_ref[...].astype(bf),
                   preferred_element_type=jnp.float32)
         + b1_ref[...])
    h = a * jax.nn.sigmoid(a)
    o_ref[...] = (jnp.dot(h.astype(bf), w2_ref[...].astype(bf),
                          preferred_element_type=jnp.float32)
                  + b2_ref[...])


def _tc_mlp(g, feat, w1h, w1f, b1, w2, b2, gam, bet, fuse_ln, block_e=1280):
    ne = g.shape[0]
    grid = ne // block_e
    row2 = lambda i: (0, 0)
    return pl.pallas_call(
        functools.partial(_mlp_block, fuse_ln=fuse_ln),
        grid=(grid,),
        in_specs=[
            pl.BlockSpec((block_e, _DP), lambda i: (i, 0)),
            pl.BlockSpec((block_e, _EF), lambda i: (i, 0)),
            pl.BlockSpec((_D, _D), row2),
            pl.BlockSpec((_EF, _D), row2),
            pl.BlockSpec((1, _D), row2),
            pl.BlockSpec((_D, _D), row2),
            pl.BlockSpec((1, _D), row2),
            pl.BlockSpec((1, _D), row2),
            pl.BlockSpec((1, _D), row2),
        ],
        out_specs=pl.BlockSpec((block_e, _D), lambda i: (i, 0)),
        out_shape=jax.ShapeDtypeStruct((ne, _D), jnp.float32),
        name="tc_edge_mlp",
    )(g, feat, w1h, w1f, b1, w2, b2, gam, bet)


def _ln2_block(x_ref, y_ref, gs_ref, bes_ref, gr_ref, ber_ref,
               ox_ref, oy_ref):
    ox_ref[...] = _layernorm(x_ref[...], gs_ref[...], bes_ref[...])
    oy_ref[...] = _layernorm(y_ref[...], gr_ref[...], ber_ref[...])


def _tc_layernorm2(x, y, gs, bes, gr, ber, block_n=2000):
    n = x.shape[0]
    grid = n // block_n
    blk = pl.BlockSpec((block_n, _D), lambda i: (i, 0))
    vec = pl.BlockSpec((1, _D), lambda i: (0, 0))
    return pl.pallas_call(
        _ln2_block,
        grid=(grid,),
        in_specs=[blk, blk, vec, vec, vec, vec],
        out_specs=[blk, blk],
        out_shape=[jax.ShapeDtypeStruct((n, _D), jnp.float32),
                   jax.ShapeDtypeStruct((y.shape[0], _D), jnp.float32)],
        name="tc_layernorm2",
    )(x, y, gs, bes, gr, ber)


def kernel(h_species, h_reactions, rs_feat, sr_feat,
           W1_rs, b1_rs, W2_rs, b2_rs, g_s, be_s,
           W1_sr, b1_sr, W2_sr, b2_sr, g_r, be_r,
           rs_index, sr_index):
    w1h_rs, w1f_rs = W1_rs[:_D], W1_rs[_D:]
    w1h_sr, w1f_sr = W1_sr[:_D], W1_sr[_D:]
    b1_rs2, b2_rs2 = b1_rs[None, :], b2_rs[None, :]
    b1_sr2, b2_sr2 = b1_sr[None, :], b2_sr[None, :]
    g_s2, be_s2 = g_s[None, :], be_s[None, :]
    g_r2, be_r2 = g_r[None, :], be_r[None, :]

    rs_src = rs_index[0].reshape(_NW, _GN, _GC)
    sr_src = sr_index[0].reshape(_NW, _GN, _GC)
    rs_dst = rs_index[1].reshape(_NSUB, _SN, _SC)
    sr_dst = sr_index[1].reshape(_NSUB, _SN, _SC)

    def _pack(x):
        x16 = x.astype(jnp.bfloat16).reshape(x.shape[0], _DP, 2)
        return jax.lax.bitcast_convert_type(x16, jnp.float32)

    perm = jnp.arange(_D)
    perm = 2 * (perm % _DP) + perm // _DP

    g1 = _sc_gather(_pack(h_reactions), rs_src)
    msgs1 = _tc_mlp(g1, rs_feat, w1h_rs[perm], w1f_rs, b1_rs2, W2_rs, b2_rs2,
                    g_s2, be_s2, fuse_ln=False)
    s_raw = _sc_scatter_add(msgs1, rs_dst, h_species)

    g2 = _sc_gather(_pack(s_raw), sr_src)
    msgs2 = _tc_mlp(g2, sr_feat, w1h_sr[perm], w1f_sr, b1_sr2, W2_sr, b2_sr2,
                    g_s[perm][None, :], be_s[perm][None, :], fuse_ln=True)
    r_raw = _sc_scatter_add(msgs2, sr_dst, h_reactions)

    h_species_out, h_reactions_out = _tc_layernorm2(
        s_raw, r_raw, g_s2, be_s2, g_r2, be_r2)
    return (h_species_out, h_reactions_out)


# speedup vs baseline: 1.0787x; 1.0787x over previous
import functools

import jax
import jax.numpy as jnp
from jax import lax
from jax.experimental import pallas as pl
from jax.experimental.pallas import tpu as pltpu
from jax.experimental.pallas import tpu_sc as plsc

_NS = 10000
_NR = 10000
_E = 160000
_D = 256
_EF = 16

_NC = 2
_NSUB = 16
_NW = _NC * _NSUB

_NBG = 5
_NBS = 2
_GC = 40
_GN = _E // _NW // _GC
_SC = 80
_SN = _E // _NSUB // _SC
_HALF = _D // _NC


@functools.cache
def _mesh():
    return plsc.VectorSubcoreMesh(core_axis_name="c", subcore_axis_name="s")


def _pipe(n, nb, fire_a, drain_a, fire_b, drain_b):
    if n % nb:
        assert nb == 2 and n % 2 == 1
        fire_a(0, 0)
        fire_a(1, 1)

        @pl.loop(2, n - 1, step=2)
        def _pipe_odd(ch0):
            for b in (0, 1):
                ch = ch0 + b
                drain_a(b)
                fire_b(ch - 2, b)
                drain_b(b)
                fire_a(ch, b)

        drain_a(0)
        fire_b(n - 3, 0)
        drain_b(0)
        fire_a(n - 1, 0)
        drain_a(1)
        fire_b(n - 2, 1)
        drain_a(0)
        fire_b(n - 1, 0)
        drain_b(1)
        drain_b(0)
        return

    assert n >= 2 * nb
    for b in range(nb):
        fire_a(b, b)

    @pl.loop(nb, n, step=nb)
    def _body(ch0):
        for b in range(nb):
            drain_a(b)
            fire_b(ch0 - nb + b, b)
        for b in range(nb):
            drain_b(b)
            fire_a(ch0 + b, b)

    for b in range(nb):
        drain_a(b)
        fire_b(n - nb + b, b)
    for b in range(nb):
        drain_b(b)


def _gather_body(table_hbm, idx_hbm, out_hbm, idx_v, buf, *sems):
    c = lax.axis_index("c")
    s = lax.axis_index("s")
    wid = s * _NC + c
    e0 = wid * (_GN * _GC)
    pltpu.sync_copy(idx_hbm.at[wid], idx_v)
    gsem = sems[:_NBG]
    wsem = sems[_NBG:]

    def fire_gather(ch, b):
        pltpu.async_copy(table_hbm.at[idx_v.at[ch]], buf.at[b], gsem[b])

    def drain_gather(b):
        pltpu.make_async_copy(table_hbm.at[idx_v.at[0]], buf.at[b],
                              gsem[b]).wait()

    def fire_wb(ch, b):
        base = pl.multiple_of(e0 + ch * _GC, 8)
        pltpu.async_copy(buf.at[b], out_hbm.at[pl.ds(base, _GC)], wsem[b])

    def drain_wb(b):
        pltpu.make_async_copy(buf.at[b], out_hbm.at[pl.ds(0, _GC)],
                              wsem[b]).wait()

    _pipe(_GN, _NBG, fire_gather, drain_gather, fire_wb, drain_wb)


def _sc_gather(table, idx3):
    return pl.kernel(
        _gather_body,
        out_type=jax.ShapeDtypeStruct((_E, _D), jnp.float32),
        mesh=_mesh(),
        scratch_types=[
            pltpu.VMEM((_GN, _GC), jnp.int32),
            pltpu.VMEM((_NBG, _GC, _D), jnp.float32),
        ] + [pltpu.SemaphoreType.DMA] * (2 * _NBG),
        name="sc_gather_rows",
    )(table, idx3)


def _scatter_body(msgs_hbm, idx_hbm, base_hbm, out_hbm, didx_v, mbuf, acc_sh,
                  *sems):
    c = lax.axis_index("c")
    s = lax.axis_index("s")
    col0 = pl.multiple_of(c * _HALF, _HALF)
    lsem = sems[:_NBS]
    ssem = sems[_NBS:]

    pltpu.sync_copy(idx_hbm.at[s], didx_v)
    n_rows = base_hbm.shape[0]
    nrc = n_rows // _SC
    my_rows = (nrc - s + _NSUB - 1) // _NSUB

    def init_body(i, carry):
        r0 = pl.multiple_of((s + i * _NSUB) * _SC, 8)
        pltpu.sync_copy(base_hbm.at[pl.ds(r0, _SC), pl.ds(col0, _HALF)],
                        mbuf.at[0])
        pltpu.sync_copy(mbuf.at[0], acc_sh.at[pl.ds(r0, _SC)])
        return carry

    lax.fori_loop(0, my_rows, init_body, 0)
    plsc.subcore_barrier()

    e0 = s * (_E // _NSUB)

    def fire_load(ch, b):
        base = pl.multiple_of(e0 + ch * _SC, 8)
        pltpu.async_copy(msgs_hbm.at[pl.ds(base, _SC), pl.ds(col0, _HALF)],
                         mbuf.at[b], lsem[b])

    def drain_load(b):
        pltpu.make_async_copy(
            msgs_hbm.at[pl.ds(0, _SC), pl.ds(col0, _HALF)], mbuf.at[b],
            lsem[b]).wait()

    def fire_scat(ch, b):
        pltpu.async_copy(mbuf.at[b], acc_sh.at[didx_v.at[ch]], ssem[b],
                         add=True)

    def drain_scat(b):
        pltpu.make_async_copy(mbuf.at[b], acc_sh.at[didx_v.at[0]],
                              ssem[b]).wait()

    _pipe(_SN, _NBS, fire_load, drain_load, fire_scat, drain_scat)
    plsc.subcore_barrier()

    def out_body(i, carry):
        r0 = pl.multiple_of((s + i * _NSUB) * _SC, 8)
        pltpu.sync_copy(acc_sh.at[pl.ds(r0, _SC)], mbuf.at[0])
        pltpu.sync_copy(mbuf.at[0],
                        out_hbm.at[pl.ds(r0, _SC), pl.ds(col0, _HALF)])
        return carry

    lax.fori_loop(0, my_rows, out_body, 0)


def _sc_scatter_add(msgs, idx3, base):
    n = base.shape[0]
    return pl.kernel(
        _scatter_body,
        out_type=jax.ShapeDtypeStruct((n, _D), jnp.float32),
        mesh=_mesh(),
        scratch_types=[
            pltpu.VMEM((_SN, _SC), jnp.int32),
            pltpu.VMEM((_NBS, _SC, _HALF), jnp.float32),
            pltpu.VMEM_SHARED((n, _HALF), jnp.float32),
        ] + [pltpu.SemaphoreType.DMA] * (2 * _NBS),
        name="sc_scatter_add",
    )(msgs, idx3, base)


def _layernorm(x, g, b, eps=1e-5):
    mu = jnp.mean(x, axis=-1, keepdims=True)
    var = jnp.mean((x - mu) ** 2, axis=-1, keepdims=True)
    return (x - mu) / jnp.sqrt(var + eps) * g + b


def _mlp_block(g_ref, f_ref, w1h_ref, w1f_ref, b1_ref, w2_ref, b2_ref,
               gam_ref, bet_ref, o_ref, *, fuse_ln):
    x = g_ref[...]
    if fuse_ln:
        x = _layernorm(x, gam_ref[...], bet_ref[...])
    a = (jnp.dot(x, w1h_ref[...], preferred_element_type=jnp.float32)
         + jnp.dot(f_ref[...], w1f_ref[...], preferred_element_type=jnp.float32)
         + b1_ref[...])
    h = a * jax.nn.sigmoid(a)
    o_ref[...] = (jnp.dot(h, w2_ref[...], preferred_element_type=jnp.float32)
                  + b2_ref[...])


def _tc_mlp(g, feat, w1h, w1f, b1, w2, b2, gam, bet, fuse_ln, block_e=1280):
    ne = g.shape[0]
    grid = ne // block_e
    row2 = lambda i: (0, 0)
    return pl.pallas_call(
        functools.partial(_mlp_block, fuse_ln=fuse_ln),
        grid=(grid,),
        in_specs=[
            pl.BlockSpec((block_e, _D), lambda i: (i, 0)),
            pl.BlockSpec((block_e, _EF), lambda i: (i, 0)),
            pl.BlockSpec((_D, _D), row2),
            pl.BlockSpec((_EF, _D), row2),
            pl.BlockSpec((1, _D), row2),
            pl.BlockSpec((_D, _D), row2),
            pl.BlockSpec((1, _D), row2),
            pl.BlockSpec((1, _D), row2),
            pl.BlockSpec((1, _D), row2),
        ],
        out_specs=pl.BlockSpec((block_e, _D), lambda i: (i, 0)),
        out_shape=jax.ShapeDtypeStruct((ne, _D), jnp.float32),
        name="tc_edge_mlp",
    )(g, feat, w1h, w1f, b1, w2, b2, gam, bet)


def _ln2_block(x_ref, y_ref, gs_ref, bes_ref, gr_ref, ber_ref,
               ox_ref, oy_ref):
    ox_ref[...] = _layernorm(x_ref[...], gs_ref[...], bes_ref[...])
    oy_ref[...] = _layernorm(y_ref[...], gr_ref[...], ber_ref[...])


def _tc_layernorm2(x, y, gs, bes, gr, ber, block_n=2000):
    n = x.shape[0]
    grid = n // block_n
    blk = pl.BlockSpec((block_n, _D), lambda i: (i, 0))
    vec = pl.BlockSpec((1, _D), lambda i: (0, 0))
    return pl.pallas_call(
        _ln2_block,
        grid=(grid,),
        in_specs=[blk, blk, vec, vec, vec, vec],
        out_specs=[blk, blk],
        out_shape=[jax.ShapeDtypeStruct((n, _D), jnp.float32),
                   jax.ShapeDtypeStruct((y.shape[0], _D), jnp.float32)],
        name="tc_layernorm2",
    )(x, y, gs, bes, gr, ber)


def kernel(h_species, h_reactions, rs_feat, sr_feat,
           W1_rs, b1_rs, W2_rs, b2_rs, g_s, be_s,
           W1_sr, b1_sr, W2_sr, b2_sr, g_r, be_r,
           rs_index, sr_index):
    w1h_rs, w1f_rs = W1_rs[:_D], W1_rs[_D:]
    w1h_sr, w1f_sr = W1_sr[:_D], W1_sr[_D:]
    b1_rs2, b2_rs2 = b1_rs[None, :], b2_rs[None, :]
    b1_sr2, b2_sr2 = b1_sr[None, :], b2_sr[None, :]
    g_s2, be_s2 = g_s[None, :], be_s[None, :]
    g_r2, be_r2 = g_r[None, :], be_r[None, :]

    rs_src = rs_index[0].reshape(_NW, _GN, _GC)
    sr_src = sr_index[0].reshape(_NW, _GN, _GC)
    rs_dst = rs_index[1].reshape(_NSUB, _SN, _SC)
    sr_dst = sr_index[1].reshape(_NSUB, _SN, _SC)

    g1 = _sc_gather(h_reactions, rs_src)
    msgs1 = _tc_mlp(g1, rs_feat, w1h_rs, w1f_rs, b1_rs2, W2_rs, b2_rs2,
                    g_s2, be_s2, fuse_ln=False)
    s_raw = _sc_scatter_add(msgs1, rs_dst, h_species)

    g2 = _sc_gather(s_raw, sr_src)
    msgs2 = _tc_mlp(g2, sr_feat, w1h_sr, w1f_sr, b1_sr2, W2_sr, b2_sr2,
                    g_s2, be_s2, fuse_ln=True)
    r_raw = _sc_scatter_add(msgs2, sr_dst, h_reactions)

    h_species_out, h_reactions_out = _tc_layernorm2(
        s_raw, r_raw, g_s2, be_s2, g_r2, be_r2)
    return (h_species_out, h_reactions_out)


# speedup vs baseline: 1.1966x; 1.1093x over previous
import functools

import jax
import jax.numpy as jnp
from jax import lax
from jax.experimental import pallas as pl
from jax.experimental.pallas import tpu as pltpu
from jax.experimental.pallas import tpu_sc as plsc

_NS = 10000
_NR = 10000
_E = 160000
_D = 256
_EF = 16

_NC = 2
_NSUB = 16
_NW = _NC * _NSUB

_NBG = 5
_NBS = 2
_GC = 40
_GN = _E // _NW // _GC
_SC = 80
_SN = _E // _NSUB // _SC
_HALF = _D // _NC


@functools.cache
def _mesh():
    return plsc.VectorSubcoreMesh(core_axis_name="c", subcore_axis_name="s")


def _pipe(n, nb, fire_a, drain_a, fire_b, drain_b):
    if n % nb:
        assert nb == 2 and n % 2 == 1
        fire_a(0, 0)
        fire_a(1, 1)

        @pl.loop(2, n - 1, step=2)
        def _pipe_odd(ch0):
            for b in (0, 1):
                ch = ch0 + b
                drain_a(b)
                fire_b(ch - 2, b)
                drain_b(b)
                fire_a(ch, b)

        drain_a(0)
        fire_b(n - 3, 0)
        drain_b(0)
        fire_a(n - 1, 0)
        drain_a(1)
        fire_b(n - 2, 1)
        drain_a(0)
        fire_b(n - 1, 0)
        drain_b(1)
        drain_b(0)
        return

    assert n >= 2 * nb
    for b in range(nb):
        fire_a(b, b)

    @pl.loop(nb, n, step=nb)
    def _body(ch0):
        for b in range(nb):
            drain_a(b)
            fire_b(ch0 - nb + b, b)
        for b in range(nb):
            drain_b(b)
            fire_a(ch0 + b, b)

    for b in range(nb):
        drain_a(b)
        fire_b(n - nb + b, b)
    for b in range(nb):
        drain_b(b)


def _gather_body(table_hbm, idx_hbm, out_hbm, idx_v, buf, *sems):
    c = lax.axis_index("c")
    s = lax.axis_index("s")
    wid = s * _NC + c
    e0 = wid * (_GN * _GC)
    pltpu.sync_copy(idx_hbm.at[wid], idx_v)
    gsem = sems[:_NBG]
    wsem = sems[_NBG:]

    def fire_gather(ch, b):
        pltpu.async_copy(table_hbm.at[idx_v.at[ch]], buf.at[b], gsem[b])

    def drain_gather(b):
        pltpu.make_async_copy(table_hbm.at[idx_v.at[0]], buf.at[b],
                              gsem[b]).wait()

    def fire_wb(ch, b):
        base = pl.multiple_of(e0 + ch * _GC, 8)
        pltpu.async_copy(buf.at[b], out_hbm.at[pl.ds(base, _GC)], wsem[b])

    def drain_wb(b):
        pltpu.make_async_copy(buf.at[b], out_hbm.at[pl.ds(0, _GC)],
                              wsem[b]).wait()

    _pipe(_GN, _NBG, fire_gather, drain_gather, fire_wb, drain_wb)


def _sc_gather(table, idx3):
    return pl.kernel(
        _gather_body,
        out_type=jax.ShapeDtypeStruct((_E, _D), jnp.float32),
        mesh=_mesh(),
        scratch_types=[
            pltpu.VMEM((_GN, _GC), jnp.int32),
            pltpu.VMEM((_NBG, _GC, _D), jnp.float32),
        ] + [pltpu.SemaphoreType.DMA] * (2 * _NBG),
        name="sc_gather_rows",
    )(table, idx3)


def _scatter_body(msgs_hbm, idx_hbm, base_hbm, out_hbm, didx_v, mbuf, acc_sh,
                  *sems):
    c = lax.axis_index("c")
    s = lax.axis_index("s")
    col0 = pl.multiple_of(c * _HALF, _HALF)
    lsem = sems[:_NBS]
    ssem = sems[_NBS:]

    pltpu.sync_copy(idx_hbm.at[s], didx_v)
    n_rows = base_hbm.shape[0]
    nrc = n_rows // _SC
    my_rows = (nrc - s + _NSUB - 1) // _NSUB

    def init_body(i, carry):
        r0 = pl.multiple_of((s + i * _NSUB) * _SC, 8)
        pltpu.sync_copy(base_hbm.at[pl.ds(r0, _SC), pl.ds(col0, _HALF)],
                        mbuf.at[0])
        pltpu.sync_copy(mbuf.at[0], acc_sh.at[pl.ds(r0, _SC)])
        return carry

    lax.fori_loop(0, my_rows, init_body, 0)
    plsc.subcore_barrier()

    e0 = s * (_E // _NSUB)

    def fire_load(ch, b):
        base = pl.multiple_of(e0 + ch * _SC, 8)
        pltpu.async_copy(msgs_hbm.at[pl.ds(base, _SC), pl.ds(col0, _HALF)],
                         mbuf.at[b], lsem[b])

    def drain_load(b):
        pltpu.make_async_copy(
            msgs_hbm.at[pl.ds(0, _SC), pl.ds(col0, _HALF)], mbuf.at[b],
            lsem[b]).wait()

    def fire_scat(ch, b):
        pltpu.async_copy(mbuf.at[b], acc_sh.at[didx_v.at[ch]], ssem[b],
                         add=True)

    def drain_scat(b):
        pltpu.make_async_copy(mbuf.at[b], acc_sh.at[didx_v.at[0]],
                              ssem[b]).wait()

    _pipe(_SN, _NBS, fire_load, drain_load, fire_scat, drain_scat)
    plsc.subcore_barrier()

    def out_body(i, carry):
        r0 = pl.multiple_of((s + i * _NSUB) * _SC, 8)
        pltpu.sync_copy(acc_sh.at[pl.ds(r0, _SC)], mbuf.at[0])
        pltpu.sync_copy(mbuf.at[0],
                        out_hbm.at[pl.ds(r0, _SC), pl.ds(col0, _HALF)])
        return carry

    lax.fori_loop(0, my_rows, out_body, 0)


def _sc_scatter_add(msgs, idx3, base):
    n = base.shape[0]
    return pl.kernel(
        _scatter_body,
        out_type=jax.ShapeDtypeStruct((n, _D), jnp.float32),
        mesh=_mesh(),
        scratch_types=[
            pltpu.VMEM((_SN, _SC), jnp.int32),
            pltpu.VMEM((_NBS, _SC, _HALF), jnp.float32),
            pltpu.VMEM_SHARED((n, _HALF), jnp.float32),
        ] + [pltpu.SemaphoreType.DMA] * (2 * _NBS),
        name="sc_scatter_add",
    )(msgs, idx3, base)


def _layernorm(x, g, b, eps=1e-5):
    mu = jnp.mean(x, axis=-1, keepdims=True)
    var = jnp.mean((x - mu) ** 2, axis=-1, keepdims=True)
    return (x - mu) / jnp.sqrt(var + eps) * g + b


def _mlp_block(g_ref, f_ref, w1h_ref, w1f_ref, b1_ref, w2_ref, b2_ref,
               gam_ref, bet_ref, o_ref, *, fuse_ln):
    x = g_ref[...]
    if fuse_ln:
        x = _layernorm(x, gam_ref[...], bet_ref[...])
    a = (jnp.dot(x, w1h_ref[...], preferred_element_type=jnp.float32)
         + jnp.dot(f_ref[...], w1f_ref[...], preferred_element_type=jnp.float32)
         + b1_ref[...])
    h = a * jax.nn.sigmoid(a)
    o_ref[...] = (jnp.dot(h, w2_ref[...], preferred_element_type=jnp.float32)
                  + b2_ref[...])


def _tc_mlp(g, feat, w1h, w1f, b1, w2, b2, gam, bet, fuse_ln, block_e=3200):
    ne = g.shape[0]
    grid = ne // block_e
    row2 = lambda i: (0, 0)
    return pl.pallas_call(
        functools.partial(_mlp_block, fuse_ln=fuse_ln),
        grid=(grid,),
        in_specs=[
            pl.BlockSpec((block_e, _D), lambda i: (i, 0)),
            pl.BlockSpec((block_e, _EF), lambda i: (i, 0)),
            pl.BlockSpec((_D, _D), row2),
            pl.BlockSpec((_EF, _D), row2),
            pl.BlockSpec((1, _D), row2),
            pl.BlockSpec((_D, _D), row2),
            pl.BlockSpec((1, _D), row2),
            pl.BlockSpec((1, _D), row2),
            pl.BlockSpec((1, _D), row2),
        ],
        out_specs=pl.BlockSpec((block_e, _D), lambda i: (i, 0)),
        out_shape=jax.ShapeDtypeStruct((ne, _D), jnp.float32),
        name="tc_edge_mlp",
    )(g, feat, w1h, w1f, b1, w2, b2, gam, bet)


def _ln2_block(x_ref, y_ref, gs_ref, bes_ref, gr_ref, ber_ref,
               ox_ref, oy_ref):
    ox_ref[...] = _layernorm(x_ref[...], gs_ref[...], bes_ref[...])
    oy_ref[...] = _layernorm(y_ref[...], gr_ref[...], ber_ref[...])


def _tc_layernorm2(x, y, gs, bes, gr, ber, block_n=2000):
    n = x.shape[0]
    grid = n // block_n
    blk = pl.BlockSpec((block_n, _D), lambda i: (i, 0))
    vec = pl.BlockSpec((1, _D), lambda i: (0, 0))
    return pl.pallas_call(
        _ln2_block,
        grid=(grid,),
        in_specs=[blk, blk, vec, vec, vec, vec],
        out_specs=[blk, blk],
        out_shape=[jax.ShapeDtypeStruct((n, _D), jnp.float32),
                   jax.ShapeDtypeStruct((y.shape[0], _D), jnp.float32)],
        name="tc_layernorm2",
    )(x, y, gs, bes, gr, ber)


def kernel(h_species, h_reactions, rs_feat, sr_feat,
           W1_rs, b1_rs, W2_rs, b2_rs, g_s, be_s,
           W1_sr, b1_sr, W2_sr, b2_sr, g_r, be_r,
           rs_index, sr_index):
    w1h_rs, w1f_rs = W1_rs[:_D], W1_rs[_D:]
    w1h_sr, w1f_sr = W1_sr[:_D], W1_sr[_D:]
    b1_rs2, b2_rs2 = b1_rs[None, :], b2_rs[None, :]
    b1_sr2, b2_sr2 = b1_sr[None, :], b2_sr[None, :]
    g_s2, be_s2 = g_s[None, :], be_s[None, :]
    g_r2, be_r2 = g_r[None, :], be_r[None, :]

    rs_src = rs_index[0].reshape(_NW, _GN, _GC)
    sr_src = sr_index[0].reshape(_NW, _GN, _GC)
    rs_dst = rs_index[1].reshape(_NSUB, _SN, _SC)
    sr_dst = sr_index[1].reshape(_NSUB, _SN, _SC)

    g1 = _sc_gather(h_reactions, rs_src)
    msgs1 = _tc_mlp(g1, rs_feat, w1h_rs, w1f_rs, b1_rs2, W2_rs, b2_rs2,
                    g_s2, be_s2, fuse_ln=False)
    s_raw = _sc_scatter_add(msgs1, rs_dst, h_species)

    g2 = _sc_gather(s_raw, sr_src)
    msgs2 = _tc_mlp(g2, sr_feat, w1h_sr, w1f_sr, b1_sr2, W2_sr, b2_sr2,
                    g_s2, be_s2, fuse_ln=True)
    r_raw = _sc_scatter_add(msgs2, sr_dst, h_reactions)

    h_species_out, h_reactions_out = _tc_layernorm2(
        s_raw, r_raw, g_s2, be_s2, g_r2, be_r2)
    return (h_species_out, h_reactions_out)


# speedup vs baseline: 1.2221x; 1.0213x over previous
import functools

import jax
import jax.numpy as jnp
from jax import lax
from jax.experimental import pallas as pl
from jax.experimental.pallas import tpu as pltpu
from jax.experimental.pallas import tpu_sc as plsc

_NS = 10000
_NR = 10000
_E = 160000
_D = 256
_EF = 16

_NC = 2
_NSUB = 16
_NW = _NC * _NSUB

_NBG = 5
_NBS = 2
_GC = 40
_GN = _E // _NW // _GC
_SC = 80
_SN = _E // _NSUB // _SC
_HALF = _D // _NC


@functools.cache
def _mesh():
    return plsc.VectorSubcoreMesh(core_axis_name="c", subcore_axis_name="s")


def _pipe(n, nb, fire_a, drain_a, fire_b, drain_b):
    if n % nb:
        assert nb == 2 and n % 2 == 1
        fire_a(0, 0)
        fire_a(1, 1)

        @pl.loop(2, n - 1, step=2)
        def _pipe_odd(ch0):
            for b in (0, 1):
                ch = ch0 + b
                drain_a(b)
                fire_b(ch - 2, b)
                drain_b(b)
                fire_a(ch, b)

        drain_a(0)
        fire_b(n - 3, 0)
        drain_b(0)
        fire_a(n - 1, 0)
        drain_a(1)
        fire_b(n - 2, 1)
        drain_a(0)
        fire_b(n - 1, 0)
        drain_b(1)
        drain_b(0)
        return

    assert n >= 2 * nb
    for b in range(nb):
        fire_a(b, b)

    @pl.loop(nb, n, step=nb)
    def _body(ch0):
        for b in range(nb):
            drain_a(b)
            fire_b(ch0 - nb + b, b)
        for b in range(nb):
            drain_b(b)
            fire_a(ch0 + b, b)

    for b in range(nb):
        drain_a(b)
        fire_b(n - nb + b, b)
    for b in range(nb):
        drain_b(b)


def _gather_body(table_hbm, idx_hbm, out_hbm, idx_v, buf, *sems):
    c = lax.axis_index("c")
    s = lax.axis_index("s")
    wid = s * _NC + c
    e0 = wid * (_GN * _GC)
    pltpu.sync_copy(idx_hbm.at[wid], idx_v)
    gsem = sems[:_NBG]
    wsem = sems[_NBG:]

    def fire_gather(ch, b):
        pltpu.async_copy(table_hbm.at[idx_v.at[ch]], buf.at[b], gsem[b])

    def drain_gather(b):
        pltpu.make_async_copy(table_hbm.at[idx_v.at[0]], buf.at[b],
                              gsem[b]).wait()

    def fire_wb(ch, b):
        base = pl.multiple_of(e0 + ch * _GC, 8)
        pltpu.async_copy(buf.at[b], out_hbm.at[pl.ds(base, _GC)], wsem[b])

    def drain_wb(b):
        pltpu.make_async_copy(buf.at[b], out_hbm.at[pl.ds(0, _GC)],
                              wsem[b]).wait()

    _pipe(_GN, _NBG, fire_gather, drain_gather, fire_wb, drain_wb)


def _sc_gather(table, idx3):
    return pl.kernel(
        _gather_body,
        out_type=jax.ShapeDtypeStruct((_E, _D), jnp.float32),
        mesh=_mesh(),
        scratch_types=[
            pltpu.VMEM((_GN, _GC), jnp.int32),
            pltpu.VMEM((_NBG, _GC, _D), jnp.float32),
        ] + [pltpu.SemaphoreType.DMA] * (2 * _NBG),
        name="sc_gather_rows",
    )(table, idx3)


def _scatter_body(msgs_hbm, idx_hbm, base_hbm, out_hbm, didx_v, mbuf, acc_sh,
                  *sems):
    c = lax.axis_index("c")
    s = lax.axis_index("s")
    col0 = pl.multiple_of(c * _HALF, _HALF)
    lsem = sems[:_NBS]
    ssem = sems[_NBS:]

    pltpu.sync_copy(idx_hbm.at[s], didx_v)
    n_rows = base_hbm.shape[0]
    nrc = n_rows // _SC
    my_rows = (nrc - s + _NSUB - 1) // _NSUB

    def init_body(i, carry):
        r0 = pl.multiple_of((s + i * _NSUB) * _SC, 8)
        pltpu.sync_copy(base_hbm.at[pl.ds(r0, _SC), pl.ds(col0, _HALF)],
                        mbuf.at[0])
        pltpu.sync_copy(mbuf.at[0], acc_sh.at[pl.ds(r0, _SC)])
        return carry

    lax.fori_loop(0, my_rows, init_body, 0)
    plsc.subcore_barrier()

    e0 = s * (_E // _NSUB)

    def fire_load(ch, b):
        base = pl.multiple_of(e0 + ch * _SC, 8)
        pltpu.async_copy(msgs_hbm.at[pl.ds(base, _SC), pl.ds(col0, _HALF)],
                         mbuf.at[b], lsem[b])

    def drain_load(b):
        pltpu.make_async_copy(
            msgs_hbm.at[pl.ds(0, _SC), pl.ds(col0, _HALF)], mbuf.at[b],
            lsem[b]).wait()

    def fire_scat(ch, b):
        pltpu.async_copy(mbuf.at[b], acc_sh.at[didx_v.at[ch]], ssem[b],
                         add=True)

    def drain_scat(b):
        pltpu.make_async_copy(mbuf.at[b], acc_sh.at[didx_v.at[0]],
                              ssem[b]).wait()

    _pipe(_SN, _NBS, fire_load, drain_load, fire_scat, drain_scat)
    plsc.subcore_barrier()

    def out_body(i, carry):
        r0 = pl.multiple_of((s + i * _NSUB) * _SC, 8)
        pltpu.sync_copy(acc_sh.at[pl.ds(r0, _SC)], mbuf.at[0])
        pltpu.sync_copy(mbuf.at[0],
                        out_hbm.at[pl.ds(r0, _SC), pl.ds(col0, _HALF)])
        return carry

    lax.fori_loop(0, my_rows, out_body, 0)


def _sc_scatter_add(msgs, idx3, base):
    n = base.shape[0]
    return pl.kernel(
        _scatter_body,
        out_type=jax.ShapeDtypeStruct((n, _D), jnp.float32),
        mesh=_mesh(),
        scratch_types=[
            pltpu.VMEM((_SN, _SC), jnp.int32),
            pltpu.VMEM((_NBS, _SC, _HALF), jnp.float32),
            pltpu.VMEM_SHARED((n, _HALF), jnp.float32),
        ] + [pltpu.SemaphoreType.DMA] * (2 * _NBS),
        name="sc_scatter_add",
    )(msgs, idx3, base)


def _layernorm(x, g, b, eps=1e-5):
    mu = jnp.mean(x, axis=-1, keepdims=True)
    var = jnp.mean((x - mu) ** 2, axis=-1, keepdims=True)
    return (x - mu) / jnp.sqrt(var + eps) * g + b


def _mlp_block(g_ref, f_ref, w1h_ref, w1f_ref, b1_ref, w2_ref, b2_ref,
               gam_ref, bet_ref, o_ref, *, fuse_ln):
    x = g_ref[...]
    if fuse_ln:
        x = _layernorm(x, gam_ref[...], bet_ref[...])
    a = (jnp.dot(x, w1h_ref[...], preferred_element_type=jnp.float32)
         + jnp.dot(f_ref[...], w1f_ref[...], preferred_element_type=jnp.float32)
         + b1_ref[...])
    h = a * jax.nn.sigmoid(a)
    o_ref[...] = (jnp.dot(h, w2_ref[...], preferred_element_type=jnp.float32)
                  + b2_ref[...])


def _tc_mlp(g, feat, w1h, w1f, b1, w2, b2, gam, bet, fuse_ln, block_e=6400):
    ne = g.shape[0]
    grid = ne // block_e
    row2 = lambda i: (0, 0)
    return pl.pallas_call(
        functools.partial(_mlp_block, fuse_ln=fuse_ln),
        grid=(grid,),
        in_specs=[
            pl.BlockSpec((block_e, _D), lambda i: (i, 0)),
            pl.BlockSpec((block_e, _EF), lambda i: (i, 0)),
            pl.BlockSpec((_D, _D), row2),
            pl.BlockSpec((_EF, _D), row2),
            pl.BlockSpec((1, _D), row2),
            pl.BlockSpec((_D, _D), row2),
            pl.BlockSpec((1, _D), row2),
            pl.BlockSpec((1, _D), row2),
            pl.BlockSpec((1, _D), row2),
        ],
        out_specs=pl.BlockSpec((block_e, _D), lambda i: (i, 0)),
        out_shape=jax.ShapeDtypeStruct((ne, _D), jnp.float32),
        name="tc_edge_mlp",
    )(g, feat, w1h, w1f, b1, w2, b2, gam, bet)


def _ln2_block(x_ref, y_ref, gs_ref, bes_ref, gr_ref, ber_ref,
               ox_ref, oy_ref):
    ox_ref[...] = _layernorm(x_ref[...], gs_ref[...], bes_ref[...])
    oy_ref[...] = _layernorm(y_ref[...], gr_ref[...], ber_ref[...])


def _tc_layernorm2(x, y, gs, bes, gr, ber, block_n=2000):
    n = x.shape[0]
    grid = n // block_n
    blk = pl.BlockSpec((block_n, _D), lambda i: (i, 0))
    vec = pl.BlockSpec((1, _D), lambda i: (0, 0))
    return pl.pallas_call(
        _ln2_block,
        grid=(grid,),
        in_specs=[blk, blk, vec, vec, vec, vec],
        out_specs=[blk, blk],
        out_shape=[jax.ShapeDtypeStruct((n, _D), jnp.float32),
                   jax.ShapeDtypeStruct((y.shape[0], _D), jnp.float32)],
        name="tc_layernorm2",
    )(x, y, gs, bes, gr, ber)


def kernel(h_species, h_reactions, rs_feat, sr_feat,
           W1_rs, b1_rs, W2_rs, b2_rs, g_s, be_s,
           W1_sr, b1_sr, W2_sr, b2_sr, g_r, be_r,
           rs_index, sr_index):
    w1h_rs, w1f_rs = W1_rs[:_D], W1_rs[_D:]
    w1h_sr, w1f_sr = W1_sr[:_D], W1_sr[_D:]
    b1_rs2, b2_rs2 = b1_rs[None, :], b2_rs[None, :]
    b1_sr2, b2_sr2 = b1_sr[None, :], b2_sr[None, :]
    g_s2, be_s2 = g_s[None, :], be_s[None, :]
    g_r2, be_r2 = g_r[None, :], be_r[None, :]

    rs_src = rs_index[0].reshape(_NW, _GN, _GC)
    sr_src = sr_index[0].reshape(_NW, _GN, _GC)
    rs_dst = rs_index[1].reshape(_NSUB, _SN, _SC)
    sr_dst = sr_index[1].reshape(_NSUB, _SN, _SC)

    g1 = _sc_gather(h_reactions, rs_src)
    msgs1 = _tc_mlp(g1, rs_feat, w1h_rs, w1f_rs, b1_rs2, W2_rs, b2_rs2,
                    g_s2, be_s2, fuse_ln=False)
    s_raw = _sc_scatter_add(msgs1, rs_dst, h_species)

    g2 = _sc_gather(s_raw, sr_src)
    msgs2 = _tc_mlp(g2, sr_feat, w1h_sr, w1f_sr, b1_sr2, W2_sr, b2_sr2,
                    g_s2, be_s2, fuse_ln=True)
    r_raw = _sc_scatter_add(msgs2, sr_dst, h_reactions)

    h_species_out, h_reactions_out = _tc_layernorm2(
        s_raw, r_raw, g_s2, be_s2, g_r2, be_r2)
    return (h_species_out, h_reactions_out)
